# baseline (speedup 1.0000x reference)
import functools

import jax
import jax.numpy as jnp
from jax.experimental import pallas as pl
from jax.experimental.pallas import tpu as pltpu

N = 10000
DEG = 32
D = 128
OUT = 128
BN = 200
G = BN // 8


def _sublane_reduce8(P, op):
    i2 = jax.lax.broadcasted_iota(jnp.int32, (1, 1, 8, 1), 2)
    m4 = i2 < 4
    m2 = (i2 & 2) == 0
    m1 = (i2 & 1) == 0
    r = op(P, pltpu.roll(P, 4, axis=2))
    m = jnp.where(m4, r[:, 0:4], r[:, 4:8])
    r = op(m, jnp.where(m2, pltpu.roll(m, 6, axis=2),
                        pltpu.roll(m, 2, axis=2)))
    m = jnp.where(m2, r[:, 0:2], r[:, 2:4])
    r = op(m, jnp.where(m1, pltpu.roll(m, 7, axis=2),
                        pltpu.roll(m, 1, axis=2)))
    out = jnp.where(m1, r[:, 0], r[:, 1])
    return out


def _pna_kernel(h_ref, nf_ref, w_ref, b_ref, out_ref):
    inv = 1.0 / DEG
    hb = h_ref[...].reshape(BN, DEG // 8, 8, D)
    t0 = hb[:, 0]
    s4 = t0
    q4 = t0 * t0
    mn4 = t0
    mx4 = t0
    for t in range(1, DEG // 8):
        x = hb[:, t]
        s4 = s4 + x
        q4 = q4 + x * x
        mn4 = jnp.minimum(mn4, x)
        mx4 = jnp.maximum(mx4, x)
    add = lambda a, b: a + b
    s = _sublane_reduce8(s4.reshape(G, 8, 8, D), add).reshape(BN, D)
    q = _sublane_reduce8(q4.reshape(G, 8, 8, D), add).reshape(BN, D)
    mn = _sublane_reduce8(mn4.reshape(G, 8, 8, D), jnp.minimum).reshape(BN, D)
    mx = _sublane_reduce8(mx4.reshape(G, 8, 8, D), jnp.maximum).reshape(BN, D)
    mean = s * inv
    var = q * inv - mean * mean
    std = jnp.sqrt(jax.nn.relu(var) + 1e-5)
    w = w_ref[...]
    acc = jnp.dot(mean, w[0:D], preferred_element_type=jnp.float32)
    acc += jnp.dot(mn, w[D:2 * D], preferred_element_type=jnp.float32)
    acc += jnp.dot(mx, w[2 * D:3 * D], preferred_element_type=jnp.float32)
    acc += jnp.dot(std, w[3 * D:4 * D], preferred_element_type=jnp.float32)
    acc += jnp.dot(nf_ref[...], w[4 * D:5 * D],
                   preferred_element_type=jnp.float32)
    out_ref[...] = acc + b_ref[...]


@jax.jit
def kernel(h, node_feat, W, b):
    b2 = b.reshape(1, OUT)
    grid = (N // BN,)
    return pl.pallas_call(
        _pna_kernel,
        grid=grid,
        in_specs=[
            pl.BlockSpec((BN, DEG, D), lambda i: (i, 0, 0)),
            pl.BlockSpec((BN, D), lambda i: (i, 0)),
            pl.BlockSpec((5 * D, OUT), lambda i: (0, 0)),
            pl.BlockSpec((1, OUT), lambda i: (0, 0)),
        ],
        out_specs=pl.BlockSpec((BN, OUT), lambda i: (i, 0)),
        out_shape=jax.ShapeDtypeStruct((N, OUT), jnp.float32),
    )(h, node_feat, W, b2)


# speedup vs baseline: 1.3351x; 1.3351x over previous
import functools

import jax
import jax.numpy as jnp
from jax.experimental import pallas as pl
from jax.experimental.pallas import tpu as pltpu

N = 10000
DEG = 32
D = 128
OUT = 128
BN = 1000
G = BN // 8


def _sublane_reduce8(P, op):
    i2 = jax.lax.broadcasted_iota(jnp.int32, (1, 1, 8, 1), 2)
    m4 = i2 < 4
    m2 = (i2 & 2) == 0
    m1 = (i2 & 1) == 0
    r = op(P, pltpu.roll(P, 4, axis=2))
    m = jnp.where(m4, r[:, 0:4], r[:, 4:8])
    r = op(m, jnp.where(m2, pltpu.roll(m, 6, axis=2),
                        pltpu.roll(m, 2, axis=2)))
    m = jnp.where(m2, r[:, 0:2], r[:, 2:4])
    r = op(m, jnp.where(m1, pltpu.roll(m, 7, axis=2),
                        pltpu.roll(m, 1, axis=2)))
    out = jnp.where(m1, r[:, 0], r[:, 1])
    return out


def _pna_kernel(h_ref, nf_ref, w_ref, b_ref, out_ref):
    inv = 1.0 / DEG
    hb = h_ref[...].reshape(BN, DEG // 8, 8, D)
    t0 = hb[:, 0]
    s4 = t0
    q4 = t0 * t0
    mn4 = t0
    mx4 = t0
    for t in range(1, DEG // 8):
        x = hb[:, t]
        s4 = s4 + x
        q4 = q4 + x * x
        mn4 = jnp.minimum(mn4, x)
        mx4 = jnp.maximum(mx4, x)
    add = lambda a, b: a + b
    s = _sublane_reduce8(s4.reshape(G, 8, 8, D), add).reshape(BN, D)
    q = _sublane_reduce8(q4.reshape(G, 8, 8, D), add).reshape(BN, D)
    mn = _sublane_reduce8(mn4.reshape(G, 8, 8, D), jnp.minimum).reshape(BN, D)
    mx = _sublane_reduce8(mx4.reshape(G, 8, 8, D), jnp.maximum).reshape(BN, D)
    mean = s * inv
    var = q * inv - mean * mean
    std = jnp.sqrt(jax.nn.relu(var) + 1e-5)
    w = w_ref[...]
    acc = jnp.dot(mean, w[0:D], preferred_element_type=jnp.float32)
    acc += jnp.dot(mn, w[D:2 * D], preferred_element_type=jnp.float32)
    acc += jnp.dot(mx, w[2 * D:3 * D], preferred_element_type=jnp.float32)
    acc += jnp.dot(std, w[3 * D:4 * D], preferred_element_type=jnp.float32)
    acc += jnp.dot(nf_ref[...], w[4 * D:5 * D],
                   preferred_element_type=jnp.float32)
    out_ref[...] = acc + b_ref[...]


@jax.jit
def kernel(h, node_feat, W, b):
    b2 = b.reshape(1, OUT)
    grid = (N // BN,)
    return pl.pallas_call(
        _pna_kernel,
        grid=grid,
        in_specs=[
            pl.BlockSpec((BN, DEG, D), lambda i: (i, 0, 0)),
            pl.BlockSpec((BN, D), lambda i: (i, 0)),
            pl.BlockSpec((5 * D, OUT), lambda i: (0, 0)),
            pl.BlockSpec((1, OUT), lambda i: (0, 0)),
        ],
        out_specs=pl.BlockSpec((BN, OUT), lambda i: (i, 0)),
        out_shape=jax.ShapeDtypeStruct((N, OUT), jnp.float32),
    )(h, node_feat, W, b2)
